# Initial kernel scaffold; baseline (speedup 1.0000x reference)
import jax
import jax.numpy as jnp
from jax.experimental import pallas as pl


def kernel(input):
    raise NotImplementedError("write your pallas kernel here")


# speedup vs baseline: 1.0431x; 1.0431x over previous
import jax
import jax.numpy as jnp
from jax.experimental import pallas as pl

N = 256
OUT_W = N * (N + 1) // 2
BATCH_BLK = 8


def _seg_off(i):
    return i * N - i * (i - 1) // 2


def _body(x_ref, o_ref):
    for i in range(N):
        m = N - i
        o_ref[:, pl.ds(_seg_off(i), m)] = x_ref[:, pl.ds(i * (N + 1), m)]


def kernel(input):
    B = input.shape[0]
    x2 = input.reshape(B, N * N)
    out = pl.pallas_call(
        _body,
        grid=(B // BATCH_BLK,),
        in_specs=[pl.BlockSpec((BATCH_BLK, N * N), lambda b: (b, 0))],
        out_specs=pl.BlockSpec((BATCH_BLK, OUT_W), lambda b: (b, 0)),
        out_shape=jax.ShapeDtypeStruct((B, OUT_W), input.dtype),
    )(x2)
    return out


# speedup vs baseline: 2.3041x; 2.2090x over previous
import jax
import jax.numpy as jnp
from jax.experimental import pallas as pl

N = 256
OUT_W = N * (N + 1) // 2
BATCH_BLK = 8


def _seg_off(i):
    return i * N - i * (i - 1) // 2


def _body(x_ref, o_ref):
    for tr in range(N // 8):
        blk = jnp.swapaxes(x_ref[:, 8 * tr : 8 * tr + 8, :], 0, 1)
        for s in range(8):
            i = 8 * tr + s
            m = N - i
            o_ref[:, pl.ds(_seg_off(i), m)] = blk[s, :, i:]


def kernel(input):
    B = input.shape[0]
    out = pl.pallas_call(
        _body,
        grid=(B // BATCH_BLK,),
        in_specs=[pl.BlockSpec((BATCH_BLK, N, N), lambda b: (b, 0, 0))],
        out_specs=pl.BlockSpec((BATCH_BLK, OUT_W), lambda b: (b, 0)),
        out_shape=jax.ShapeDtypeStruct((B, OUT_W), input.dtype),
    )(input)
    return out


# speedup vs baseline: 2.4621x; 1.0685x over previous
import jax
import jax.numpy as jnp
from jax.experimental import pallas as pl

N = 256
OUT_W = N * (N + 1) // 2
BATCH_BLK = 8
H = N // 2


def _seg_off(i):
    return i * N - i * (i - 1) // 2


def _body(xa_ref, xb_ref, o_ref):
    for tr in range(N // 8):
        if tr < H // 8:
            blk = jnp.swapaxes(xa_ref[:, 8 * tr : 8 * tr + 8, :], 0, 1)
        else:
            blk = jnp.swapaxes(
                xb_ref[:, 8 * tr - H : 8 * tr - H + 8, :], 0, 1
            )
        for s in range(8):
            i = 8 * tr + s
            m = N - i
            col0 = i if i < H else i - H
            o_ref[:, pl.ds(_seg_off(i), m)] = blk[s, :, col0:]


def kernel(input):
    B = input.shape[0]
    out = pl.pallas_call(
        _body,
        grid=(B // BATCH_BLK,),
        in_specs=[
            pl.BlockSpec((BATCH_BLK, H, N), lambda b: (b, 0, 0)),
            pl.BlockSpec((BATCH_BLK, H, H), lambda b: (b, 1, 1)),
        ],
        out_specs=pl.BlockSpec((BATCH_BLK, OUT_W), lambda b: (b, 0)),
        out_shape=jax.ShapeDtypeStruct((B, OUT_W), input.dtype),
    )(input, input)
    return out


# speedup vs baseline: 3.2205x; 1.3081x over previous
import jax
import jax.numpy as jnp
from jax.experimental import pallas as pl

N = 256
OUT_W = N * (N + 1) // 2
BATCH_BLK = 16
H = N // 2


def _seg_off(i):
    return i * N - i * (i - 1) // 2


def _body(xa_ref, xb_ref, o_ref):
    for tr in range(N // 8):
        if tr < H // 8:
            blk = jnp.swapaxes(xa_ref[:, 8 * tr : 8 * tr + 8, :], 0, 1)
        else:
            blk = jnp.swapaxes(
                xb_ref[:, 8 * tr - H : 8 * tr - H + 8, :], 0, 1
            )
        for s in range(8):
            i = 8 * tr + s
            m = N - i
            col0 = i if i < H else i - H
            o_ref[:, pl.ds(_seg_off(i), m)] = blk[s, :, col0:]


def kernel(input):
    B = input.shape[0]
    out = pl.pallas_call(
        _body,
        grid=(B // BATCH_BLK,),
        in_specs=[
            pl.BlockSpec((BATCH_BLK, H, N), lambda b: (b, 0, 0)),
            pl.BlockSpec((BATCH_BLK, H, H), lambda b: (b, 1, 1)),
        ],
        out_specs=pl.BlockSpec((BATCH_BLK, OUT_W), lambda b: (b, 0)),
        out_shape=jax.ShapeDtypeStruct((B, OUT_W), input.dtype),
    )(input, input)
    return out


# speedup vs baseline: 3.4642x; 1.0757x over previous
import jax
import jax.numpy as jnp
from jax.experimental import pallas as pl

N = 256
OUT_W = N * (N + 1) // 2
BATCH_BLK = 32
H = N // 2


def _seg_off(i):
    return i * N - i * (i - 1) // 2


def _body(xa_ref, xb_ref, o_ref):
    for tr in range(N // 8):
        if tr < H // 8:
            blk = jnp.swapaxes(xa_ref[:, 8 * tr : 8 * tr + 8, :], 0, 1)
        else:
            blk = jnp.swapaxes(
                xb_ref[:, 8 * tr - H : 8 * tr - H + 8, :], 0, 1
            )
        for s in range(8):
            i = 8 * tr + s
            m = N - i
            col0 = i if i < H else i - H
            o_ref[:, pl.ds(_seg_off(i), m)] = blk[s, :, col0:]


def kernel(input):
    B = input.shape[0]
    out = pl.pallas_call(
        _body,
        grid=(B // BATCH_BLK,),
        in_specs=[
            pl.BlockSpec((BATCH_BLK, H, N), lambda b: (b, 0, 0)),
            pl.BlockSpec((BATCH_BLK, H, H), lambda b: (b, 1, 1)),
        ],
        out_specs=pl.BlockSpec((BATCH_BLK, OUT_W), lambda b: (b, 0)),
        out_shape=jax.ShapeDtypeStruct((B, OUT_W), input.dtype),
    )(input, input)
    return out


# speedup vs baseline: 3.5624x; 1.0284x over previous
import jax
import jax.numpy as jnp
from jax.experimental import pallas as pl

N = 256
OUT_W = N * (N + 1) // 2
BATCH_BLK = 64
H = N // 2


def _seg_off(i):
    return i * N - i * (i - 1) // 2


def _body(xa_ref, xb_ref, o_ref):
    for tr in range(N // 8):
        if tr < H // 8:
            blk = jnp.swapaxes(xa_ref[:, 8 * tr : 8 * tr + 8, :], 0, 1)
        else:
            blk = jnp.swapaxes(
                xb_ref[:, 8 * tr - H : 8 * tr - H + 8, :], 0, 1
            )
        for s in range(8):
            i = 8 * tr + s
            m = N - i
            col0 = i if i < H else i - H
            o_ref[:, pl.ds(_seg_off(i), m)] = blk[s, :, col0:]


def kernel(input):
    B = input.shape[0]
    out = pl.pallas_call(
        _body,
        grid=(B // BATCH_BLK,),
        in_specs=[
            pl.BlockSpec((BATCH_BLK, H, N), lambda b: (b, 0, 0)),
            pl.BlockSpec((BATCH_BLK, H, H), lambda b: (b, 1, 1)),
        ],
        out_specs=pl.BlockSpec((BATCH_BLK, OUT_W), lambda b: (b, 0)),
        out_shape=jax.ShapeDtypeStruct((B, OUT_W), input.dtype),
    )(input, input)
    return out
